# baseline (speedup 1.0000x reference)
import jax
import jax.numpy as jnp
from jax import lax
from jax.experimental import pallas as pl
from jax.experimental.pallas import tpu as pltpu

_N = 1000
_D = 1024
_R = 16
_K = 50
_BETA2 = 0.2 ** 2
_TAU = 0.01
_SCALING = 2.0
_G = 8
_KPAD = 56
_NG = _KPAD // _G


def _body(q_ref, cor_ref, wb_ref, a_hbm, b_hbm, out_ref,
          idx_sm, w_sm, acat, bcat, b_buf, a1d, b1d, a_sem, b_sem):
    q = q_ref[...]
    qn = jnp.sqrt(jnp.sum(q * q))
    scores = lax.dot_general(q, cor_ref[...], (((1,), (1,)), ((), ())),
                             preferred_element_type=jnp.float32)
    csq = jnp.zeros((1, _N), jnp.float32)
    ones = jnp.ones((1, 128), jnp.float32)
    for t in range(_D // 128):
        ch = cor_ref[:, 128 * t:128 * (t + 1)]
        csq = csq + lax.dot_general(ones, ch * ch, (((1,), (1,)), ((), ())),
                                    preferred_element_type=jnp.float32)
    cn = jnp.sqrt(csq)
    sim = scores / ((cn + 1e-9) * (qn + 1e-9)) / _BETA2
    mx = jnp.max(sim)
    e = jnp.exp(sim - mx)
    p = e / jnp.sum(e)
    p = jnp.where(p >= _TAU, p, 0.0)

    lane = lax.broadcasted_iota(jnp.int32, (1, _N), 1)

    def topk_body(t, carry):
        pc, s = carry
        mt = jnp.max(pc)
        it = jnp.min(jnp.where(pc == mt, lane, jnp.int32(2**30)))
        idx_sm[t] = it
        w_sm[t] = mt
        pc = jnp.where(lane == it, -1.0, pc)
        return pc, s + mt

    if True:
        ssum = jnp.sum(p)
        for t in range(_KPAD):
            idx_sm[t] = t
            w_sm[t] = ssum

    def a_copy(k):
        return pltpu.make_async_copy(
            a_hbm.at[pl.ds(idx_sm[k] * _R * _D, _R * _D)], a1d, a_sem)

    def b_copy(k, slot, j):
        return pltpu.make_async_copy(
            b_hbm.at[pl.ds(idx_sm[k] * _R * _D, _R * _D)], b1d, b_sem.at[slot])

    for k in range(1):
        a_copy(k).start()
    for k in range(1):
        a_copy(k).wait()
    bcat[...] = jnp.zeros((_D, _KPAD * _R), jnp.bfloat16)


    delta = jnp.dot(bcat[...], acat[...].astype(jnp.bfloat16),
                    preferred_element_type=jnp.float32)
    out_ref[...] = wb_ref[...] + delta


def kernel(q, corpus, A_all, B_all, W_base):
    return pl.pallas_call(
        _body,
        out_shape=jax.ShapeDtypeStruct((_D, _D), jnp.float32),
        in_specs=[
            pl.BlockSpec(memory_space=pltpu.VMEM),
            pl.BlockSpec(memory_space=pltpu.VMEM),
            pl.BlockSpec(memory_space=pltpu.VMEM),
            pl.BlockSpec(memory_space=pltpu.HBM),
            pl.BlockSpec(memory_space=pltpu.HBM),
        ],
        out_specs=pl.BlockSpec(memory_space=pltpu.VMEM),
        scratch_shapes=[
            pltpu.SMEM((_KPAD,), jnp.int32),
            pltpu.SMEM((_KPAD,), jnp.float32),
            pltpu.VMEM((_KPAD * _R, _D), jnp.float32),
            pltpu.VMEM((_D, _KPAD * _R), jnp.bfloat16),
            pltpu.VMEM((2, _G, _D, _R), jnp.float32),
            pltpu.VMEM((_R * _D,), jnp.float32),
            pltpu.VMEM((_R * _D,), jnp.float32),
            pltpu.SemaphoreType.DMA,
            pltpu.SemaphoreType.DMA((2,)),
        ],
    )(q, corpus, W_base, A_all.reshape(-1), B_all.reshape(-1))


# speedup vs baseline: 15.1556x; 15.1556x over previous
import jax
import jax.numpy as jnp
from jax import lax
from jax.experimental import pallas as pl
from jax.experimental.pallas import tpu as pltpu

_N = 1000
_D = 1024
_R = 16
_K = 50
_BETA2 = 0.2 ** 2
_TAU = 0.01
_SCALING = 2.0
_KPAD = 56


def _body(q_ref, cor_ref, wb_ref, a_hbm, bt_hbm, out_ref,
          idx_sm, w_sm, acat, bcat, a_sem, b_sem):
    q = q_ref[...]
    qn = jnp.sqrt(jnp.sum(q * q))
    scores = lax.dot_general(q, cor_ref[...], (((1,), (1,)), ((), ())),
                             preferred_element_type=jnp.float32)
    csq = jnp.zeros((1, _N), jnp.float32)
    ones = jnp.ones((1, 128), jnp.float32)
    for t in range(_D // 128):
        ch = cor_ref[:, 128 * t:128 * (t + 1)]
        csq = csq + lax.dot_general(ones, ch * ch, (((1,), (1,)), ((), ())),
                                    preferred_element_type=jnp.float32)
    cn = jnp.sqrt(csq)
    sim = scores / ((cn + 1e-9) * (qn + 1e-9)) / _BETA2
    mx = jnp.max(sim)
    e = jnp.exp(sim - mx)
    p = e / jnp.sum(e)
    p = jnp.where(p >= _TAU, p, 0.0)

    lane = lax.broadcasted_iota(jnp.int32, (1, _N), 1)

    def topk_body(t, carry):
        pc, s = carry
        mt = jnp.max(pc)
        it = jnp.min(jnp.where(pc == mt, lane, jnp.int32(2**30)))
        idx_sm[t] = it
        w_sm[t] = mt
        pc = jnp.where(lane == it, -1.0, pc)
        return pc, s + mt

    _, ssum = lax.fori_loop(0, _K, topk_body, (p, jnp.float32(0.0)))
    wscale = _SCALING / (ssum + 1e-9)

    def a_copy(k):
        return pltpu.make_async_copy(
            a_hbm.at[idx_sm[k]], acat.at[pl.ds(k * _R, _R), :], a_sem)

    def b_copy(k):
        return pltpu.make_async_copy(
            bt_hbm.at[idx_sm[k]], bcat.at[pl.ds(k * _R, _R), :], b_sem)

    for k in range(_K):
        a_copy(k).start()
        b_copy(k).start()
    zpad = jnp.zeros(((_KPAD - _K) * _R, _D), jnp.float32)
    bcat[pl.ds(_K * _R, (_KPAD - _K) * _R), :] = zpad
    acat[pl.ds(_K * _R, (_KPAD - _K) * _R), :] = zpad

    lane64 = lax.broadcasted_iota(jnp.int32, (1, 64), 1)
    wrow = jnp.zeros((1, 64), jnp.float32)
    for k in range(_K):
        wrow = jnp.where(lane64 == k, w_sm[k] * wscale, wrow)
    rowg = lax.broadcasted_iota(jnp.int32, (_KPAD * _R, 64), 0) // _R
    kcol = lax.broadcasted_iota(jnp.int32, (_KPAD * _R, 64), 1)
    eye_g = jnp.where(rowg == kcol, 1.0, 0.0)
    scale_col = lax.dot_general(eye_g, wrow, (((1,), (1,)), ((), ())),
                                preferred_element_type=jnp.float32)

    for k in range(_K):
        b_copy(k).wait()
    for k in range(_K):
        a_copy(k).wait()

    delta = lax.dot_general(
        (bcat[...] * scale_col).astype(jnp.bfloat16),
        acat[...].astype(jnp.bfloat16),
        (((0,), (0,)), ((), ())), preferred_element_type=jnp.float32)
    out_ref[...] = wb_ref[...] + delta


def kernel(q, corpus, A_all, B_all, W_base):
    B_t = jnp.swapaxes(B_all, 1, 2)
    return pl.pallas_call(
        _body,
        out_shape=jax.ShapeDtypeStruct((_D, _D), jnp.float32),
        in_specs=[
            pl.BlockSpec(memory_space=pltpu.VMEM),
            pl.BlockSpec(memory_space=pltpu.VMEM),
            pl.BlockSpec(memory_space=pltpu.VMEM),
            pl.BlockSpec(memory_space=pltpu.HBM),
            pl.BlockSpec(memory_space=pltpu.HBM),
        ],
        out_specs=pl.BlockSpec(memory_space=pltpu.VMEM),
        scratch_shapes=[
            pltpu.SMEM((_KPAD,), jnp.int32),
            pltpu.SMEM((_KPAD,), jnp.float32),
            pltpu.VMEM((_KPAD * _R, _D), jnp.float32),
            pltpu.VMEM((_KPAD * _R, _D), jnp.float32),
            pltpu.SemaphoreType.DMA,
            pltpu.SemaphoreType.DMA,
        ],
    )(q, corpus, W_base, A_all, B_t)


# speedup vs baseline: 22.1819x; 1.4636x over previous
import jax
import jax.numpy as jnp
from jax import lax
from jax.experimental import pallas as pl
from jax.experimental.pallas import tpu as pltpu

_N = 1000
_D = 1024
_R = 16
_K = 50
_BETA2 = 0.2 ** 2
_TAU = 0.01
_SCALING = 2.0
_KPAD = 56


def _body(q_ref, cor_ref, wb_ref, a_hbm, bt_hbm, out_ref,
          idx_sm, idx_v, acat, bcat, a_sem, b_sem, i_sem):
    q = q_ref[...]
    qn = jnp.sqrt(jnp.sum(q * q))
    scores = lax.dot_general(q, cor_ref[...], (((1,), (1,)), ((), ())),
                             preferred_element_type=jnp.float32)
    csq = jnp.zeros((1, _N), jnp.float32)
    ones = jnp.ones((1, 128), jnp.float32)
    for t in range(_D // 128):
        ch = cor_ref[:, 128 * t:128 * (t + 1)]
        csq = csq + lax.dot_general(ones, ch * ch, (((1,), (1,)), ((), ())),
                                    preferred_element_type=jnp.float32)
    cn = jnp.sqrt(csq)
    sim = scores / ((cn + 1e-9) * (qn + 1e-9)) / _BETA2
    mx = jnp.max(sim)
    e = jnp.exp(sim - mx)
    p = e / jnp.sum(e)
    p = jnp.where(p >= _TAU, p, 0.0)

    ii = lax.broadcasted_iota(jnp.int32, (_N, _N), 0)
    jj = lax.broadcasted_iota(jnp.int32, (_N, _N), 1)
    eye_n = jnp.where(ii == jj, 1.0, 0.0)
    p_col = lax.dot_general(eye_n, p, (((1,), (1,)), ((), ())),
                            preferred_element_type=jnp.float32)
    cmp = jnp.where(p_col < p, 1.0, 0.0)
    rank = lax.dot_general(cmp, jnp.ones((_N, 1), jnp.float32),
                           (((1,), (0,)), ((), ())),
                           preferred_element_type=jnp.float32)
    lane64 = lax.broadcasted_iota(jnp.int32, (1, 64), 1).astype(jnp.float32)
    onehot = jnp.where(rank == lane64, 1.0, 0.0)
    lane_f = lax.broadcasted_iota(jnp.int32, (1, _N), 1).astype(jnp.float32)
    idx_f = lax.dot_general(lane_f, onehot, (((1,), (0,)), ((), ())),
                            preferred_element_type=jnp.float32)
    wvec = lax.dot_general(p, onehot, (((1,), (0,)), ((), ())),
                           preferred_element_type=jnp.float32)
    sel = lane64 < float(_K)
    ssum = jnp.sum(jnp.where(sel, wvec, 0.0))
    wscale = _SCALING / (ssum + 1e-9)
    scale64 = jnp.where(sel, wvec * wscale, 0.0)
    idx_v[...] = jnp.clip(idx_f, 0.0, float(_N - 1)).astype(jnp.int32)
    pltpu.make_async_copy(idx_v, idx_sm, i_sem).start()
    pltpu.make_async_copy(idx_v, idx_sm, i_sem).wait()

    def a_copy(k):
        return pltpu.make_async_copy(
            a_hbm.at[idx_sm[0, k]], acat.at[pl.ds(k * _R, _R), :], a_sem)

    def b_copy(k):
        return pltpu.make_async_copy(
            bt_hbm.at[idx_sm[0, k]], bcat.at[pl.ds(k * _R, _R), :], b_sem)

    for k in range(_K):
        a_copy(k).start()
        b_copy(k).start()
    zpad = jnp.zeros(((_KPAD - _K) * _R, _D), jnp.float32)
    bcat[pl.ds(_K * _R, (_KPAD - _K) * _R), :] = zpad
    acat[pl.ds(_K * _R, (_KPAD - _K) * _R), :] = zpad

    rowg = lax.broadcasted_iota(jnp.int32, (_KPAD * _R, 64), 0) // _R
    kcol = lax.broadcasted_iota(jnp.int32, (_KPAD * _R, 64), 1)
    eye_g = jnp.where(rowg == kcol, 1.0, 0.0)
    scale_col = lax.dot_general(eye_g, scale64, (((1,), (1,)), ((), ())),
                                preferred_element_type=jnp.float32)

    for k in range(_K):
        b_copy(k).wait()
    for k in range(_K):
        a_copy(k).wait()

    delta = lax.dot_general(
        (bcat[...] * scale_col).astype(jnp.bfloat16),
        acat[...].astype(jnp.bfloat16),
        (((0,), (0,)), ((), ())), preferred_element_type=jnp.float32)
    out_ref[...] = wb_ref[...] + delta


def kernel(q, corpus, A_all, B_all, W_base):
    B_t = jnp.swapaxes(B_all, 1, 2)
    return pl.pallas_call(
        _body,
        out_shape=jax.ShapeDtypeStruct((_D, _D), jnp.float32),
        in_specs=[
            pl.BlockSpec(memory_space=pltpu.VMEM),
            pl.BlockSpec(memory_space=pltpu.VMEM),
            pl.BlockSpec(memory_space=pltpu.VMEM),
            pl.BlockSpec(memory_space=pltpu.HBM),
            pl.BlockSpec(memory_space=pltpu.HBM),
        ],
        out_specs=pl.BlockSpec(memory_space=pltpu.VMEM),
        scratch_shapes=[
            pltpu.SMEM((1, 64), jnp.int32),
            pltpu.VMEM((1, 64), jnp.int32),
            pltpu.VMEM((_KPAD * _R, _D), jnp.float32),
            pltpu.VMEM((_KPAD * _R, _D), jnp.float32),
            pltpu.SemaphoreType.DMA,
            pltpu.SemaphoreType.DMA,
            pltpu.SemaphoreType.DMA,
        ],
    )(q, corpus, W_base, A_all, B_t)
